# baseline (speedup 1.0000x reference)
import functools

import jax
import jax.numpy as jnp
from jax.experimental import pallas as pl
from jax.experimental.pallas import tpu as pltpu

WS = 16
KNN = 8
H = 224
W = 224
HC = 96
NH = H // WS
NWIN = NH * NH
P = WS * WS
BH = 8
NBLK = H // BH
WB = 7
SELKEY = 1e10
DIAGKEY = 2e10
PAD = 128


def _leaky(v):
    return jnp.where(v > 0, v, 0.05 * v)


def _conv_body(xp_ref, w_ref, b_ref, o_ref, *, cin, out_dtype):
    i = pl.program_id(0)
    acc = jnp.zeros((BH * W, HC), jnp.float32)
    for dy in range(3):
        rows = xp_ref[pl.ds(i * BH + dy, BH)]
        for dx in range(3):
            t = rows[:, dx:dx + W, :].reshape(BH * W, cin)
            acc = acc + jax.lax.dot(t.astype(jnp.bfloat16),
                                    w_ref[dy * 3 + dx],
                                    preferred_element_type=jnp.float32)
    acc = acc + b_ref[0]
    o_ref[...] = _leaky(acc).astype(out_dtype).reshape(BH, W, HC)


def _conv(xp, w9, b, out_dtype=jnp.float32):
    cin = xp.shape[-1]
    return pl.pallas_call(
        functools.partial(_conv_body, cin=cin, out_dtype=out_dtype),
        grid=(NBLK,),
        in_specs=[
            pl.BlockSpec((H + 2, W + 2, cin), lambda i: (0, 0, 0)),
            pl.BlockSpec((9, cin, HC), lambda i: (0, 0, 0)),
            pl.BlockSpec((1, HC), lambda i: (0, 0)),
        ],
        out_specs=pl.BlockSpec((BH, W, HC), lambda i: (i, 0, 0)),
        out_shape=jax.ShapeDtypeStruct((H, W, HC), out_dtype),
        compiler_params=pltpu.CompilerParams(
            dimension_semantics=("arbitrary",)),
    )(xp, w9, b)


def _gnn_body(f_ref, gpws_ref, gpb_ref, gws_ref, gb_ref,
              gfws_ref, gfwn_ref, gfb_ref, o_ref):
    jj = jax.lax.broadcasted_iota(jnp.uint32, (P, P), 1)
    ii = jax.lax.broadcasted_iota(jnp.uint32, (P, P), 0)
    diagb = ii == jj
    himask = jnp.uint32(0xFFFFFF00)

    def build_mask(fb):
        ff = fb.astype(jnp.float32)
        sq = jnp.sum(ff * ff, axis=1, keepdims=True).astype(jnp.bfloat16)
        one = jnp.ones((P, 1), jnp.bfloat16)
        am = jnp.concatenate([fb * jnp.bfloat16(-2.0), sq, one], axis=1)
        bm = jnp.concatenate([fb, one, sq], axis=1)
        d2 = jax.lax.dot_general(am, bm, (((1,), (1,)), ((), ())),
                                 preferred_element_type=jnp.float32)
        ku = jax.lax.bitcast_convert_type(d2, jnp.uint32)
        key = jax.lax.bitcast_convert_type((ku & himask) | jj, jnp.float32)
        key = jnp.where(diagb, DIAGKEY, key)
        kf = jnp.minimum(key[:, :P // 2], key[:, P // 2:])
        km = jnp.maximum(key[:, :P // 2], key[:, P // 2:])
        m = None
        for _ in range(KNN):
            m = jnp.min(kf, axis=1, keepdims=True)
            sel = kf == m
            kf = jnp.where(sel, km, kf)
            km = jnp.where(sel, SELKEY, km)
        return (key <= m).astype(jnp.bfloat16)

    def glayer(f, mask, wc, b):
        fA = jax.lax.dot(f, wc, preferred_element_type=jnp.float32)
        fn = fA[:, PAD:].astype(jnp.bfloat16)
        agg = jax.lax.dot(mask, fn, preferred_element_type=jnp.float32)
        out = fA[:, :HC] + agg[:, :HC] + b
        return _leaky(out).astype(jnp.bfloat16)

    hb = f_ref[...]
    fs = [hb[:, w * WS:(w + 1) * WS, :].reshape(P, HC) for w in range(WB)]
    masks = [build_mask(f) for f in fs]
    fs = [glayer(fs[w], masks[w], gpws_ref[...], gpb_ref[0])
          for w in range(WB)]
    for i in range(8):
        if i % 3 == 0:
            masks = [build_mask(f) for f in fs]
        fs = [glayer(fs[w], masks[w], gws_ref[i], gb_ref[i, 0])
              for w in range(WB)]
    for w in range(WB):
        agg = jax.lax.dot(masks[w], fs[w], preferred_element_type=jnp.float32)
        r = (jnp.sum(fs[w].astype(jnp.float32) * gfws_ref[...], axis=1)
             + jnp.sum(agg * gfwn_ref[...], axis=1) + gfb_ref[0])
        o_ref[0, w * P:(w + 1) * P] = jax.nn.sigmoid(r)


def _gnn(h, gp_wc, gp_b, g_wc, g_b, gf_ws, gf_wn, gf_b):
    nj = NH // WB
    return pl.pallas_call(
        _gnn_body,
        grid=(NH, nj),
        in_specs=[
            pl.BlockSpec((WS, WB * WS, HC), lambda i, j: (i, j, 0)),
            pl.BlockSpec((HC, 2 * PAD), lambda i, j: (0, 0)),
            pl.BlockSpec((1, HC), lambda i, j: (0, 0)),
            pl.BlockSpec((8, HC, 2 * PAD), lambda i, j: (0, 0, 0)),
            pl.BlockSpec((8, 1, HC), lambda i, j: (0, 0, 0)),
            pl.BlockSpec((1, HC), lambda i, j: (0, 0)),
            pl.BlockSpec((1, HC), lambda i, j: (0, 0)),
            pl.BlockSpec((1, 1), lambda i, j: (0, 0)),
        ],
        out_specs=pl.BlockSpec((1, WB * P), lambda i, j: (0, i * nj + j)),
        out_shape=jax.ShapeDtypeStruct((1, NWIN * P), jnp.float32),
        compiler_params=pltpu.CompilerParams(
            dimension_semantics=("arbitrary", "arbitrary")),
    )(h, gp_wc, gp_b, g_wc, g_b, gf_ws, gf_wn, gf_b)


def kernel(x, c1w, c1b, c2w, c2b, c3w, c3b, gp_ws, gp_wn, gp_b,
           g_ws, g_wn, g_b, gf_ws, gf_wn, gf_b):
    bf = jnp.bfloat16
    x2 = x[0, 0][:, :, None]
    h = _conv(jnp.pad(x2, ((1, 1), (1, 1), (0, 0))),
              c1w.transpose(2, 3, 1, 0).reshape(9, 1, HC).astype(bf),
              c1b[None])
    h = _conv(jnp.pad(h, ((1, 1), (1, 1), (0, 0))),
              c2w.transpose(2, 3, 1, 0).reshape(9, HC, HC).astype(bf),
              c2b[None])
    h = _conv(jnp.pad(h, ((1, 1), (1, 1), (0, 0))),
              c3w.transpose(2, 3, 1, 0).reshape(9, HC, HC).astype(bf),
              c3b[None], out_dtype=bf)
    gp_wc = (jnp.zeros((HC, 2 * PAD), jnp.float32)
             .at[:, :HC].set(gp_ws).at[:, PAD:PAD + HC].set(gp_wn * 0.125))
    g_wc = (jnp.zeros((8, HC, 2 * PAD), jnp.float32)
            .at[:, :, :HC].set(g_ws)
            .at[:, :, PAD:PAD + HC].set(g_wn * 0.125))
    out = _gnn(h,
               gp_wc.astype(bf), gp_b[None],
               g_wc.astype(bf), g_b[:, None, :],
               gf_ws.reshape(1, HC), (gf_wn * 0.125).reshape(1, HC),
               gf_b[None])
    y = (out.reshape(NH, NH, WS, WS)
         .transpose(0, 2, 1, 3).reshape(H, W))
    return y[None, None]


# speedup vs baseline: 1.3312x; 1.3312x over previous
import functools

import jax
import jax.numpy as jnp
from jax.experimental import pallas as pl
from jax.experimental.pallas import tpu as pltpu

WS = 16
KNN = 8
H = 224
W = 224
HC = 96
NH = H // WS
NWIN = NH * NH
P = WS * WS
BH = 8
NBLK = H // BH
WB = 4
SELKEY = 1e10
DIAGKEY = 2e10
PAD = 128


def _leaky(v):
    return jnp.where(v > 0, v, 0.05 * v)


def _conv_body(xp_ref, w_ref, b_ref, o_ref, *, cin, out_dtype):
    i = pl.program_id(0)
    acc = jnp.zeros((BH * W, HC), jnp.float32)
    for dy in range(3):
        rows = xp_ref[pl.ds(i * BH + dy, BH)]
        for dx in range(3):
            t = rows[:, dx:dx + W, :].reshape(BH * W, cin)
            acc = acc + jax.lax.dot(t.astype(jnp.bfloat16),
                                    w_ref[dy * 3 + dx],
                                    preferred_element_type=jnp.float32)
    acc = acc + b_ref[0]
    o_ref[...] = _leaky(acc).astype(out_dtype).reshape(BH, W, HC)


def _conv(xp, w9, b, out_dtype=jnp.float32):
    cin = xp.shape[-1]
    return pl.pallas_call(
        functools.partial(_conv_body, cin=cin, out_dtype=out_dtype),
        grid=(NBLK,),
        in_specs=[
            pl.BlockSpec((H + 2, W + 2, cin), lambda i: (0, 0, 0)),
            pl.BlockSpec((9, cin, HC), lambda i: (0, 0, 0)),
            pl.BlockSpec((1, HC), lambda i: (0, 0)),
        ],
        out_specs=pl.BlockSpec((BH, W, HC), lambda i: (i, 0, 0)),
        out_shape=jax.ShapeDtypeStruct((H, W, HC), out_dtype),
        compiler_params=pltpu.CompilerParams(
            dimension_semantics=("arbitrary",)),
    )(xp, w9, b)


def _conv3_body(xp_ref, w_ref, b_ref, o_ref):
    i = pl.program_id(0)
    acc = jnp.zeros((WS * W, HC), jnp.float32)
    for dy in range(3):
        rows = xp_ref[pl.ds(i * WS + dy, WS)]
        for dx in range(3):
            t = rows[:, dx:dx + W, :].reshape(WS * W, HC)
            acc = acc + jax.lax.dot(t.astype(jnp.bfloat16),
                                    w_ref[dy * 3 + dx],
                                    preferred_element_type=jnp.float32)
    acc = acc + b_ref[0]
    hwin = (_leaky(acc).astype(jnp.bfloat16)
            .reshape(WS, NH, WS, HC).transpose(1, 0, 2, 3)
            .reshape(NH, P, HC))
    o_ref[...] = hwin


def _conv3(xp, w9, b):
    return pl.pallas_call(
        _conv3_body,
        grid=(NH,),
        in_specs=[
            pl.BlockSpec((H + 2, W + 2, HC), lambda i: (0, 0, 0)),
            pl.BlockSpec((9, HC, HC), lambda i: (0, 0, 0)),
            pl.BlockSpec((1, HC), lambda i: (0, 0)),
        ],
        out_specs=pl.BlockSpec((NH, P, HC), lambda i: (i, 0, 0)),
        out_shape=jax.ShapeDtypeStruct((NWIN, P, HC), jnp.bfloat16),
        compiler_params=pltpu.CompilerParams(
            dimension_semantics=("arbitrary",)),
    )(xp, w9, b)


def _gnn_body(f_ref, gpws_ref, gpb_ref, gws_ref, gb_ref,
              gfws_ref, gfwn_ref, gfb_ref, o_ref):
    jj = jax.lax.broadcasted_iota(jnp.uint32, (P, P), 1)
    ii = jax.lax.broadcasted_iota(jnp.uint32, (P, P), 0)
    diagb = ii == jj
    himask = jnp.uint32(0xFFFFFF00)

    def build_mask(fb):
        ff = fb.astype(jnp.float32)
        sq = jnp.sum(ff * ff, axis=1, keepdims=True).astype(jnp.bfloat16)
        one = jnp.ones((P, 1), jnp.bfloat16)
        am = jnp.concatenate([fb * jnp.bfloat16(-2.0), sq, one], axis=1)
        bm = jnp.concatenate([fb, one, sq], axis=1)
        d2 = jax.lax.dot_general(am, bm, (((1,), (1,)), ((), ())),
                                 preferred_element_type=jnp.float32)
        ku = jax.lax.bitcast_convert_type(d2, jnp.uint32)
        key = jax.lax.bitcast_convert_type((ku & himask) | jj, jnp.float32)
        key = jnp.where(diagb, DIAGKEY, key)
        kf = jnp.minimum(key[:, :P // 2], key[:, P // 2:])
        km = jnp.maximum(key[:, :P // 2], key[:, P // 2:])
        m = None
        for _ in range(KNN):
            m = jnp.min(kf, axis=1, keepdims=True)
            sel = kf == m
            kf = jnp.where(sel, km, kf)
            km = jnp.where(sel, SELKEY, km)
        return (key <= m).astype(jnp.bfloat16)

    def glayer(f, mask, wc, b):
        fA = jax.lax.dot(f, wc, preferred_element_type=jnp.float32)
        fn = fA[:, PAD:].astype(jnp.bfloat16)
        agg = jax.lax.dot(mask, fn, preferred_element_type=jnp.float32)
        out = fA[:, :HC] + agg[:, :HC] + b
        return _leaky(out).astype(jnp.bfloat16)

    fs = [f_ref[w] for w in range(WB)]
    masks = [build_mask(f) for f in fs]
    fs = [glayer(fs[w], masks[w], gpws_ref[...], gpb_ref[0])
          for w in range(WB)]
    for i in range(8):
        if i % 3 == 0:
            masks = [build_mask(f) for f in fs]
        fs = [glayer(fs[w], masks[w], gws_ref[i], gb_ref[i, 0])
              for w in range(WB)]
    for w in range(WB):
        agg = jax.lax.dot(masks[w], fs[w], preferred_element_type=jnp.float32)
        r = (jnp.sum(fs[w].astype(jnp.float32) * gfws_ref[...], axis=1)
             + jnp.sum(agg * gfwn_ref[...], axis=1) + gfb_ref[0])
        o_ref[0, w * P:(w + 1) * P] = jax.nn.sigmoid(r)


def _gnn(fw, gp_wc, gp_b, g_wc, g_b, gf_ws, gf_wn, gf_b):
    return pl.pallas_call(
        _gnn_body,
        grid=(NWIN // WB,),
        in_specs=[
            pl.BlockSpec((WB, P, HC), lambda i: (i, 0, 0)),
            pl.BlockSpec((HC, 2 * PAD), lambda i: (0, 0)),
            pl.BlockSpec((1, HC), lambda i: (0, 0)),
            pl.BlockSpec((8, HC, 2 * PAD), lambda i: (0, 0, 0)),
            pl.BlockSpec((8, 1, HC), lambda i: (0, 0, 0)),
            pl.BlockSpec((1, HC), lambda i: (0, 0)),
            pl.BlockSpec((1, HC), lambda i: (0, 0)),
            pl.BlockSpec((1, 1), lambda i: (0, 0)),
        ],
        out_specs=pl.BlockSpec((1, WB * P), lambda i: (0, i)),
        out_shape=jax.ShapeDtypeStruct((1, NWIN * P), jnp.float32),
        compiler_params=pltpu.CompilerParams(
            dimension_semantics=("arbitrary",)),
    )(fw, gp_wc, gp_b, g_wc, g_b, gf_ws, gf_wn, gf_b)


def kernel(x, c1w, c1b, c2w, c2b, c3w, c3b, gp_ws, gp_wn, gp_b,
           g_ws, g_wn, g_b, gf_ws, gf_wn, gf_b):
    bf = jnp.bfloat16
    x2 = x[0, 0][:, :, None]
    h = _conv(jnp.pad(x2, ((1, 1), (1, 1), (0, 0))),
              c1w.transpose(2, 3, 1, 0).reshape(9, 1, HC).astype(bf),
              c1b[None])
    h = _conv(jnp.pad(h, ((1, 1), (1, 1), (0, 0))),
              c2w.transpose(2, 3, 1, 0).reshape(9, HC, HC).astype(bf),
              c2b[None])
    fw = _conv3(jnp.pad(h, ((1, 1), (1, 1), (0, 0))),
                c3w.transpose(2, 3, 1, 0).reshape(9, HC, HC).astype(bf),
                c3b[None])
    gp_wc = (jnp.zeros((HC, 2 * PAD), jnp.float32)
             .at[:, :HC].set(gp_ws).at[:, PAD:PAD + HC].set(gp_wn * 0.125))
    g_wc = (jnp.zeros((8, HC, 2 * PAD), jnp.float32)
            .at[:, :, :HC].set(g_ws)
            .at[:, :, PAD:PAD + HC].set(g_wn * 0.125))
    out = _gnn(fw,
               gp_wc.astype(bf), gp_b[None],
               g_wc.astype(bf), g_b[:, None, :],
               gf_ws.reshape(1, HC), (gf_wn * 0.125).reshape(1, HC),
               gf_b[None])
    y = (out.reshape(NH, NH, WS, WS)
         .transpose(0, 2, 1, 3).reshape(H, W))
    return y[None, None]


# speedup vs baseline: 1.3383x; 1.0054x over previous
import functools

import jax
import jax.numpy as jnp
from jax.experimental import pallas as pl
from jax.experimental.pallas import tpu as pltpu

WS = 16
KNN = 8
H = 224
W = 224
HC = 96
NH = H // WS
NWIN = NH * NH
P = WS * WS
BH = 8
NBLK = H // BH
WB = 4
SELKEY = 1e10
DIAGKEY = 2e10
PAD = 128


def _leaky(v):
    return jnp.where(v > 0, v, 0.05 * v)


def _conv_body(xp_ref, w_ref, b_ref, o_ref, *, cin):
    i = pl.program_id(0)
    acc = jnp.zeros((BH * W, HC), jnp.float32)
    for dy in range(3):
        rows = xp_ref[pl.ds(i * BH + dy, BH)]
        for dx in range(3):
            t = rows[:, dx:dx + W, :].reshape(BH * W, cin)
            acc = acc + jax.lax.dot(t.astype(jnp.bfloat16),
                                    w_ref[dy * 3 + dx],
                                    preferred_element_type=jnp.float32)
    acc = acc + b_ref[0]
    o_ref[...] = _leaky(acc).reshape(BH, W, HC)


def _conv(xp, w9, b):
    cin = xp.shape[-1]
    return pl.pallas_call(
        functools.partial(_conv_body, cin=cin),
        grid=(NBLK,),
        in_specs=[
            pl.BlockSpec((H + 2, W + 2, cin), lambda i: (0, 0, 0)),
            pl.BlockSpec((9, cin, HC), lambda i: (0, 0, 0)),
            pl.BlockSpec((1, HC), lambda i: (0, 0)),
        ],
        out_specs=pl.BlockSpec((BH, W, HC), lambda i: (i, 0, 0)),
        out_shape=jax.ShapeDtypeStruct((H, W, HC), jnp.float32),
        compiler_params=pltpu.CompilerParams(
            dimension_semantics=("arbitrary",)),
    )(xp, w9, b)


def _conv3_body(xp_ref, w_ref, b_ref, o_ref):
    i = pl.program_id(0)
    acc = jnp.zeros((WS * W, HC), jnp.float32)
    for dy in range(3):
        rows = xp_ref[pl.ds(i * WS + dy, WS)]
        for dx in range(3):
            t = rows[:, dx:dx + W, :].reshape(WS * W, HC)
            acc = acc + jax.lax.dot(t.astype(jnp.bfloat16),
                                    w_ref[dy * 3 + dx],
                                    preferred_element_type=jnp.float32)
    acc = acc + b_ref[0]
    hwin = (_leaky(acc).astype(jnp.bfloat16)
            .reshape(WS, NH, WS, HC).transpose(1, 0, 2, 3)
            .reshape(NH, P, HC))
    o_ref[...] = hwin


def _conv3(xp, w9, b):
    return pl.pallas_call(
        _conv3_body,
        grid=(NH,),
        in_specs=[
            pl.BlockSpec((H + 2, W + 2, HC), lambda i: (0, 0, 0)),
            pl.BlockSpec((9, HC, HC), lambda i: (0, 0, 0)),
            pl.BlockSpec((1, HC), lambda i: (0, 0)),
        ],
        out_specs=pl.BlockSpec((NH, P, HC), lambda i: (i, 0, 0)),
        out_shape=jax.ShapeDtypeStruct((NWIN, P, HC), jnp.bfloat16),
        compiler_params=pltpu.CompilerParams(
            dimension_semantics=("arbitrary",)),
    )(xp, w9, b)


def _gnn_body(f_ref, gpws_ref, gpb_ref, gws_ref, gb_ref,
              gfws_ref, gfwn_ref, gfb_ref, o_ref):
    jj = jax.lax.broadcasted_iota(jnp.uint32, (P, P), 1)
    ii = jax.lax.broadcasted_iota(jnp.uint32, (P, P), 0)
    diagb = ii == jj
    himask = jnp.uint32(0xFFFFFF00)

    def build_mask(fb):
        one = jnp.ones((P, 1), jnp.bfloat16)
        ff = fb.astype(jnp.float32)
        sq = jnp.sum(ff * ff, axis=1, keepdims=True).astype(jnp.bfloat16)
        am = jnp.concatenate([fb * jnp.bfloat16(-2.0), sq, one], axis=1)
        bm = jnp.concatenate([fb, one, sq], axis=1)
        d2 = jax.lax.dot_general(am, bm, (((1,), (1,)), ((), ())),
                                 preferred_element_type=jnp.float32)
        ku = jax.lax.bitcast_convert_type(d2, jnp.uint32)
        key = jax.lax.bitcast_convert_type((ku & himask) | jj, jnp.float32)
        key = jnp.where(diagb, DIAGKEY, key)
        kf = jnp.minimum(key[:, :P // 2], key[:, P // 2:])
        km = jnp.maximum(key[:, :P // 2], key[:, P // 2:])
        for _ in range(KNN - 1):
            m = jnp.min(kf, axis=1, keepdims=True)
            sel = kf == m
            kf = jnp.where(sel, km, kf)
            km = jnp.where(sel, SELKEY, km)
        m = jnp.min(kf, axis=1, keepdims=True)
        return (key <= m).astype(jnp.bfloat16)

    def glayer(f, mask, wc, b):
        fA = jax.lax.dot(f, wc, preferred_element_type=jnp.float32)
        fn = fA[:, PAD:].astype(jnp.bfloat16)
        agg = jax.lax.dot(mask, fn, preferred_element_type=jnp.float32)
        out = fA[:, :HC] + agg[:, :HC] + b
        return _leaky(out).astype(jnp.bfloat16)

    fs = [f_ref[w] for w in range(WB)]
    masks = [build_mask(f) for f in fs]
    fs = [glayer(fs[w], masks[w], gpws_ref[...], gpb_ref[0])
          for w in range(WB)]
    for i in range(8):
        if i % 3 == 0:
            masks = [build_mask(f) for f in fs]
        fs = [glayer(fs[w], masks[w], gws_ref[i], gb_ref[i, 0])
              for w in range(WB)]
    for w in range(WB):
        agg = jax.lax.dot(masks[w], fs[w], preferred_element_type=jnp.float32)
        r = (jnp.sum(fs[w].astype(jnp.float32) * gfws_ref[...], axis=1)
             + jnp.sum(agg * gfwn_ref[...], axis=1) + gfb_ref[0])
        o_ref[0, w * P:(w + 1) * P] = jax.nn.sigmoid(r)


def _gnn(fw, gp_wc, gp_b, g_wc, g_b, gf_ws, gf_wn, gf_b):
    return pl.pallas_call(
        _gnn_body,
        grid=(NWIN // WB,),
        in_specs=[
            pl.BlockSpec((WB, P, HC), lambda i: (i, 0, 0)),
            pl.BlockSpec((HC, 2 * PAD), lambda i: (0, 0)),
            pl.BlockSpec((1, HC), lambda i: (0, 0)),
            pl.BlockSpec((8, HC, 2 * PAD), lambda i: (0, 0, 0)),
            pl.BlockSpec((8, 1, HC), lambda i: (0, 0, 0)),
            pl.BlockSpec((1, HC), lambda i: (0, 0)),
            pl.BlockSpec((1, HC), lambda i: (0, 0)),
            pl.BlockSpec((1, 1), lambda i: (0, 0)),
        ],
        out_specs=pl.BlockSpec((1, WB * P), lambda i: (0, i)),
        out_shape=jax.ShapeDtypeStruct((1, NWIN * P), jnp.float32),
        compiler_params=pltpu.CompilerParams(
            dimension_semantics=("arbitrary",)),
    )(fw, gp_wc, gp_b, g_wc, g_b, gf_ws, gf_wn, gf_b)


def kernel(x, c1w, c1b, c2w, c2b, c3w, c3b, gp_ws, gp_wn, gp_b,
           g_ws, g_wn, g_b, gf_ws, gf_wn, gf_b):
    bf = jnp.bfloat16
    x2 = x[0, 0][:, :, None]
    h = _conv(jnp.pad(x2, ((1, 1), (1, 1), (0, 0))),
              c1w.transpose(2, 3, 1, 0).reshape(9, 1, HC).astype(bf),
              c1b[None])
    h = _conv(jnp.pad(h, ((1, 1), (1, 1), (0, 0))),
              c2w.transpose(2, 3, 1, 0).reshape(9, HC, HC).astype(bf),
              c2b[None])
    fw = _conv3(jnp.pad(h, ((1, 1), (1, 1), (0, 0))),
                c3w.transpose(2, 3, 1, 0).reshape(9, HC, HC).astype(bf),
                c3b[None])
    zpad = jnp.zeros((HC, PAD - HC), jnp.float32)
    gp_wc = jnp.concatenate([gp_ws, zpad, gp_wn * 0.125, zpad], axis=1)
    zpad8 = jnp.zeros((8, HC, PAD - HC), jnp.float32)
    g_wc = jnp.concatenate([g_ws, zpad8, g_wn * 0.125, zpad8], axis=2)
    out = _gnn(fw,
               gp_wc.astype(bf), gp_b[None],
               g_wc.astype(bf), g_b[:, None, :],
               gf_ws.reshape(1, HC), (gf_wn * 0.125).reshape(1, HC),
               gf_b[None])
    y = (out.reshape(NH, NH, WS, WS)
         .transpose(0, 2, 1, 3).reshape(H, W))
    return y[None, None]


# speedup vs baseline: 1.3972x; 1.0440x over previous
import functools

import jax
import jax.numpy as jnp
from jax.experimental import pallas as pl
from jax.experimental.pallas import tpu as pltpu

WS = 16
KNN = 8
H = 224
W = 224
HC = 96
NH = H // WS
NWIN = NH * NH
P = WS * WS
BH = 8
NBLK = H // BH
WB = 4
SELKEY = 1e10
DIAGKEY = 2e10
PAD = 128


def _leaky(v):
    return jnp.where(v > 0, v, 0.05 * v)


def _conv1_body(pt_ref, w_ref, b_ref, o_ref):
    t = pt_ref[...].reshape(BH * W, 9).astype(jnp.bfloat16)
    acc = jax.lax.dot(t, w_ref[...], preferred_element_type=jnp.float32)
    o_ref[...] = _leaky(acc + b_ref[0]).reshape(BH, W, HC)


def _conv1(patches, w, b):
    return pl.pallas_call(
        _conv1_body,
        grid=(NBLK,),
        in_specs=[
            pl.BlockSpec((BH, W, 9), lambda i: (i, 0, 0)),
            pl.BlockSpec((9, HC), lambda i: (0, 0)),
            pl.BlockSpec((1, HC), lambda i: (0, 0)),
        ],
        out_specs=pl.BlockSpec((BH, W, HC), lambda i: (i, 0, 0)),
        out_shape=jax.ShapeDtypeStruct((H, W, HC), jnp.float32),
        compiler_params=pltpu.CompilerParams(
            dimension_semantics=("arbitrary",)),
    )(patches, w, b)


def _conv_body(xp_ref, w_ref, b_ref, o_ref, *, cin):
    i = pl.program_id(0)
    acc = jnp.zeros((BH * W, HC), jnp.float32)
    for dy in range(3):
        rows = xp_ref[pl.ds(i * BH + dy, BH)]
        for dx in range(3):
            t = rows[:, dx:dx + W, :].reshape(BH * W, cin)
            acc = acc + jax.lax.dot(t.astype(jnp.bfloat16),
                                    w_ref[dy * 3 + dx],
                                    preferred_element_type=jnp.float32)
    acc = acc + b_ref[0]
    o_ref[...] = _leaky(acc).reshape(BH, W, HC)


def _conv(xp, w9, b):
    cin = xp.shape[-1]
    return pl.pallas_call(
        functools.partial(_conv_body, cin=cin),
        grid=(NBLK,),
        in_specs=[
            pl.BlockSpec((H + 2, W + 2, cin), lambda i: (0, 0, 0)),
            pl.BlockSpec((9, cin, HC), lambda i: (0, 0, 0)),
            pl.BlockSpec((1, HC), lambda i: (0, 0)),
        ],
        out_specs=pl.BlockSpec((BH, W, HC), lambda i: (i, 0, 0)),
        out_shape=jax.ShapeDtypeStruct((H, W, HC), jnp.float32),
        compiler_params=pltpu.CompilerParams(
            dimension_semantics=("arbitrary",)),
    )(xp, w9, b)


def _conv3_body(xp_ref, w_ref, b_ref, o_ref):
    i = pl.program_id(0)
    acc = jnp.zeros((WS * W, HC), jnp.float32)
    for dy in range(3):
        rows = xp_ref[pl.ds(i * WS + dy, WS)]
        for dx in range(3):
            t = rows[:, dx:dx + W, :].reshape(WS * W, HC)
            acc = acc + jax.lax.dot(t.astype(jnp.bfloat16),
                                    w_ref[dy * 3 + dx],
                                    preferred_element_type=jnp.float32)
    acc = acc + b_ref[0]
    hwin = (_leaky(acc).astype(jnp.bfloat16)
            .reshape(WS, NH, WS, HC).transpose(1, 0, 2, 3)
            .reshape(NH, P, HC))
    o_ref[...] = hwin


def _conv3(xp, w9, b):
    return pl.pallas_call(
        _conv3_body,
        grid=(NH,),
        in_specs=[
            pl.BlockSpec((H + 2, W + 2, HC), lambda i: (0, 0, 0)),
            pl.BlockSpec((9, HC, HC), lambda i: (0, 0, 0)),
            pl.BlockSpec((1, HC), lambda i: (0, 0)),
        ],
        out_specs=pl.BlockSpec((NH, P, HC), lambda i: (i, 0, 0)),
        out_shape=jax.ShapeDtypeStruct((NWIN, P, HC), jnp.bfloat16),
        compiler_params=pltpu.CompilerParams(
            dimension_semantics=("arbitrary",)),
    )(xp, w9, b)


def _gnn_body(f_ref, gpws_ref, gpb_ref, gws_ref, gb_ref,
              gfws_ref, gfwn_ref, gfb_ref, o_ref):
    jj = jax.lax.broadcasted_iota(jnp.uint32, (P, P), 1)
    ii = jax.lax.broadcasted_iota(jnp.uint32, (P, P), 0)
    diagb = ii == jj
    himask = jnp.uint32(0xFFFFFF00)

    def build_mask(fb):
        one = jnp.ones((P, 1), jnp.bfloat16)
        ff = fb.astype(jnp.float32)
        sq = jnp.sum(ff * ff, axis=1, keepdims=True).astype(jnp.bfloat16)
        am = jnp.concatenate([fb * jnp.bfloat16(-2.0), sq, one], axis=1)
        bm = jnp.concatenate([fb, one, sq], axis=1)
        d2 = jax.lax.dot_general(am, bm, (((1,), (1,)), ((), ())),
                                 preferred_element_type=jnp.float32)
        ku = jax.lax.bitcast_convert_type(d2, jnp.uint32)
        key = jax.lax.bitcast_convert_type((ku & himask) | jj, jnp.float32)
        key = jnp.where(diagb, DIAGKEY, key)
        kf = jnp.minimum(key[:, :P // 2], key[:, P // 2:])
        km = jnp.maximum(key[:, :P // 2], key[:, P // 2:])
        for _ in range(KNN - 1):
            m = jnp.min(kf, axis=1, keepdims=True)
            sel = kf == m
            kf = jnp.where(sel, km, kf)
            km = jnp.where(sel, SELKEY, km)
        m = jnp.min(kf, axis=1, keepdims=True)
        return (key <= m).astype(jnp.bfloat16)

    def glayer(f, mask, wc, b):
        fA = jax.lax.dot(f, wc, preferred_element_type=jnp.float32)
        fn = fA[:, PAD:].astype(jnp.bfloat16)
        agg = jax.lax.dot(mask, fn, preferred_element_type=jnp.float32)
        out = fA[:, :HC] + agg[:, :HC] + b
        return _leaky(out).astype(jnp.bfloat16)

    fs = [f_ref[w] for w in range(WB)]
    masks = [build_mask(f) for f in fs]
    fs = [glayer(fs[w], masks[w], gpws_ref[...], gpb_ref[0])
          for w in range(WB)]
    for i in range(8):
        if i % 3 == 0:
            masks = [build_mask(f) for f in fs]
        fs = [glayer(fs[w], masks[w], gws_ref[i], gb_ref[i, 0])
              for w in range(WB)]
    for w in range(WB):
        agg = jax.lax.dot(masks[w], fs[w], preferred_element_type=jnp.float32)
        r = (jnp.sum(fs[w].astype(jnp.float32) * gfws_ref[...], axis=1)
             + jnp.sum(agg * gfwn_ref[...], axis=1) + gfb_ref[0])
        o_ref[0, w * P:(w + 1) * P] = jax.nn.sigmoid(r)


def _gnn(fw, gp_wc, gp_b, g_wc, g_b, gf_ws, gf_wn, gf_b):
    return pl.pallas_call(
        _gnn_body,
        grid=(NWIN // WB,),
        in_specs=[
            pl.BlockSpec((WB, P, HC), lambda i: (i, 0, 0)),
            pl.BlockSpec((HC, 2 * PAD), lambda i: (0, 0)),
            pl.BlockSpec((1, HC), lambda i: (0, 0)),
            pl.BlockSpec((8, HC, 2 * PAD), lambda i: (0, 0, 0)),
            pl.BlockSpec((8, 1, HC), lambda i: (0, 0, 0)),
            pl.BlockSpec((1, HC), lambda i: (0, 0)),
            pl.BlockSpec((1, HC), lambda i: (0, 0)),
            pl.BlockSpec((1, 1), lambda i: (0, 0)),
        ],
        out_specs=pl.BlockSpec((1, WB * P), lambda i: (0, i)),
        out_shape=jax.ShapeDtypeStruct((1, NWIN * P), jnp.float32),
        compiler_params=pltpu.CompilerParams(
            dimension_semantics=("arbitrary",)),
    )(fw, gp_wc, gp_b, g_wc, g_b, gf_ws, gf_wn, gf_b)


def kernel(x, c1w, c1b, c2w, c2b, c3w, c3b, gp_ws, gp_wn, gp_b,
           g_ws, g_wn, g_b, gf_ws, gf_wn, gf_b):
    bf = jnp.bfloat16
    xp2 = jnp.pad(x[0, 0], ((1, 1), (1, 1)))
    patches = jnp.stack([xp2[dy:dy + H, dx:dx + W]
                         for dy in range(3) for dx in range(3)], axis=-1)
    h = _conv1(patches,
               c1w.transpose(2, 3, 1, 0).reshape(9, HC).astype(bf),
               c1b[None])
    h = _conv(jnp.pad(h, ((1, 1), (1, 1), (0, 0))),
              c2w.transpose(2, 3, 1, 0).reshape(9, HC, HC).astype(bf),
              c2b[None])
    fw = _conv3(jnp.pad(h, ((1, 1), (1, 1), (0, 0))),
                c3w.transpose(2, 3, 1, 0).reshape(9, HC, HC).astype(bf),
                c3b[None])
    zpad = jnp.zeros((HC, PAD - HC), jnp.float32)
    gp_wc = jnp.concatenate([gp_ws, zpad, gp_wn * 0.125, zpad], axis=1)
    zpad8 = jnp.zeros((8, HC, PAD - HC), jnp.float32)
    g_wc = jnp.concatenate([g_ws, zpad8, g_wn * 0.125, zpad8], axis=2)
    out = _gnn(fw,
               gp_wc.astype(bf), gp_b[None],
               g_wc.astype(bf), g_b[:, None, :],
               gf_ws.reshape(1, HC), (gf_wn * 0.125).reshape(1, HC),
               gf_b[None])
    y = (out.reshape(NH, NH, WS, WS)
         .transpose(0, 2, 1, 3).reshape(H, W))
    return y[None, None]
